# baseline (speedup 1.0000x reference)
import functools

import jax
import jax.numpy as jnp
from jax import lax
from jax.experimental import pallas as pl
from jax.experimental.pallas import tpu as pltpu
from jax.experimental.pallas import tpu_sc as plsc

N = 10000
D = 128
DEG = 32
K = 5
GROUP = 16
NGROUPS = N // GROUP
NWORKERS = 32
ROW_BLK = 2000


def _score_body(f_ref, wat_ref, watat_ref, s_ref):
    fb = f_ref[...].astype(jnp.bfloat16)
    wb = wat_ref[...].astype(jnp.bfloat16)
    zb = lax.dot_general(fb, wb, (((1,), (1,)), ((), ())),
                         preferred_element_type=jnp.float32)
    zbb = zb.astype(jnp.bfloat16)
    w1b = watat_ref[:, :D].astype(jnp.bfloat16)
    s_ref[...] = lax.dot_general(w1b, zbb, (((1,), (1,)), ((), ())),
                                 preferred_element_type=jnp.float32)


def _tc_score(features, W_at, W_atat):
    return pl.pallas_call(
        _score_body,
        out_shape=jax.ShapeDtypeStruct((1, N), jnp.float32),
    )(features, W_at, W_atat)


MAXI = (NGROUPS + NWORKERS - 1) // NWORKERS


def _sc_body(f_hbm, s_hbm, ei_hbm, h_hbm,
             s_v, src_v, svals_v, idx_v, rows_v, hsum_v, sems, sems_w,
             sems_src):
    cid = lax.axis_index("c")
    sid = lax.axis_index("s")
    wid = sid * 2 + cid
    iota = lax.iota(jnp.int32, GROUP)
    ioff = iota * DEG
    minf = jnp.full((GROUP,), -jnp.inf, jnp.float32)

    CH = GROUP * DEG

    def _chunk_copy(b, g):
        return pltpu.make_async_copy(ei_hbm.at[0, pl.ds(g * CH, CH)],
                                     src_v[b], sems_src[b])

    _chunk_copy(0, wid).start()
    pltpu.sync_copy(s_hbm.at[0], s_v)

    def select_and_fire(b, g):
        _chunk_copy(b, g).wait()
        gn = g + NWORKERS

        @pl.when(gn < NGROUPS)
        def _():
            _chunk_copy(1 - b, gn).start()


        zero_ai = jnp.zeros((GROUP,), jnp.int32)

        def fillmax(j, carry):
            m, ai = carry
            srcs = plsc.load_gather(src_v[b], [ioff + j])
            sv = plsc.load_gather(s_v, [srcs])
            svals_v[pl.ds(j * GROUP, GROUP)] = sv
            gt = sv > m
            return jnp.where(gt, sv, m), jnp.where(gt, j, ai)
        m, ai = lax.fori_loop(0, DEG, fillmax, (minf, zero_ai), unroll=8)

        for p in range(K):
            plsc.store_scatter(svals_v, [ai * GROUP + iota], minf)
            srcsel = plsc.load_gather(src_v[b], [ioff + ai])
            idx_v[b][pl.ds(p * GROUP, GROUP)] = srcsel
            if p == K - 1:
                break

            def scanmax(j, carry):
                m, ai = carry
                v = svals_v[pl.ds(j * GROUP, GROUP)]
                gt = v > m
                return jnp.where(gt, v, m), jnp.where(gt, j, ai)
            m, ai = lax.fori_loop(0, DEG, scanmax, (minf, zero_ai),
                                  unroll=8)

        pltpu.make_async_copy(f_hbm.at[idx_v[b]], rows_v[b], sems[b]).start()

    def drain(b, g):
        pltpu.make_async_copy(f_hbm.at[idx_v[b]], rows_v[b], sems[b]).wait()

        @pl.when(g >= 2 * NWORKERS)
        def _():
            pltpu.make_async_copy(
                hsum_v[b], h_hbm.at[pl.ds((g - 2 * NWORKERS) * GROUP, GROUP)],
                sems_w[b]).wait()

        def accum(l, c):
            for d in range(D // 16):
                acc = rows_v[b][l, pl.ds(d * 16, 16)]
                for p in range(1, K):
                    acc = acc + rows_v[b][p * GROUP + l, pl.ds(d * 16, 16)]
                hsum_v[b][l, pl.ds(d * 16, 16)] = acc
            return c
        lax.fori_loop(0, GROUP, accum, 0)

        pltpu.make_async_copy(hsum_v[b],
                              h_hbm.at[pl.ds(g * GROUP, GROUP)],
                              sems_w[b]).start()

    def step(k, c):
        for bb in range(2):
            i = k * 2 + bb
            g = wid + i * NWORKERS

            @pl.when(g < NGROUPS)
            def _():
                select_and_fire(bb, g)

            gp = wid + (i - 1) * NWORKERS

            @pl.when(jnp.logical_and(i >= 1, gp < NGROUPS))
            def _():
                drain(1 - bb, gp)
        return c

    lax.fori_loop(0, MAXI // 2, step, 0)
    g_last = wid + (MAXI - 1) * NWORKERS

    @pl.when(g_last < NGROUPS)
    def _():
        drain((MAXI - 1) % 2, g_last)

    for b in range(2):
        pltpu.make_async_copy(hsum_v[b], h_hbm.at[pl.ds(0, GROUP)],
                              sems_w[b]).wait()


def _sc_gather(features, s2d, edge_index):
    mesh = plsc.VectorSubcoreMesh(core_axis_name="c", subcore_axis_name="s")
    run = pl.kernel(
        _sc_body,
        out_type=jax.ShapeDtypeStruct((N, D), jnp.float32),
        mesh=mesh,
        scratch_types=[
            pltpu.VMEM((N,), jnp.float32),
            [pltpu.VMEM((GROUP * DEG,), jnp.int32)] * 2,
            pltpu.VMEM((DEG * GROUP,), jnp.float32),
            [pltpu.VMEM((K * GROUP,), jnp.int32)] * 2,
            [pltpu.VMEM((K * GROUP, D), jnp.float32)] * 2,
            [pltpu.VMEM((GROUP, D), jnp.float32)] * 2,
            [pltpu.SemaphoreType.DMA] * 2,
            [pltpu.SemaphoreType.DMA] * 2,
            [pltpu.SemaphoreType.DMA] * 2,
        ],
        compiler_params=pltpu.CompilerParams(needs_layout_passes=False),
    )
    return run(features, s2d, edge_index)


def _out_body(f_ref, h_ref, w_ref, b_ref, o_ref):
    x = (f_ref[...] + h_ref[...] * (1.0 / K)).astype(jnp.bfloat16)
    wb = w_ref[...].astype(jnp.bfloat16)
    o_ref[...] = lax.dot_general(x, wb, (((1,), (1,)), ((), ())),
                                 preferred_element_type=jnp.float32) + b_ref[...]


def _tc_out(features, h, W_self, b_row):
    return pl.pallas_call(
        _out_body,
        grid=(N // ROW_BLK,),
        in_specs=[
            pl.BlockSpec((ROW_BLK, D), lambda i: (i, 0)),
            pl.BlockSpec((ROW_BLK, D), lambda i: (i, 0)),
            pl.BlockSpec((D, D), lambda i: (0, 0)),
            pl.BlockSpec((1, D), lambda i: (0, 0)),
        ],
        out_specs=pl.BlockSpec((ROW_BLK, D), lambda i: (i, 0)),
        out_shape=jax.ShapeDtypeStruct((N, D), jnp.float32),
    )(features, h, W_self, b_row)


@jax.jit
def _run(features, edge_index, W_at, W_atat, W_self, b_self):
    s = _tc_score(features, W_at, W_atat)
    h = _sc_gather(features, s, edge_index)
    return _tc_out(features, h, W_self, b_self.reshape(1, D))


def kernel(features, edge_index, PERSON_NUM, JOINT_NUM,
           W_at, W_atat, W_self, b_self):
    return _run(features, edge_index.astype(jnp.int32),
                W_at, W_atat, W_self, b_self)


# speedup vs baseline: 1.0019x; 1.0019x over previous
import functools

import jax
import jax.numpy as jnp
from jax import lax
from jax.experimental import pallas as pl
from jax.experimental.pallas import tpu as pltpu
from jax.experimental.pallas import tpu_sc as plsc

N = 10000
D = 128
DEG = 32
K = 5
GROUP = 16
NGROUPS = N // GROUP
NWORKERS = 32
ROW_BLK = 2000


def _score_body(f_ref, wat_ref, watat_ref, s_ref):
    fb = f_ref[...].astype(jnp.bfloat16)
    wb = wat_ref[...].astype(jnp.bfloat16)
    zb = lax.dot_general(fb, wb, (((1,), (1,)), ((), ())),
                         preferred_element_type=jnp.float32)
    zbb = zb.astype(jnp.bfloat16)
    w1b = watat_ref[:, :D].astype(jnp.bfloat16)
    s_ref[...] = lax.dot_general(w1b, zbb, (((1,), (1,)), ((), ())),
                                 preferred_element_type=jnp.float32)


def _tc_score(features, W_at, W_atat):
    return pl.pallas_call(
        _score_body,
        out_shape=jax.ShapeDtypeStruct((1, N), jnp.float32),
    )(features, W_at, W_atat)


MAXI = (NGROUPS + NWORKERS - 1) // NWORKERS


def _sc_body(f_hbm, s_hbm, ei_hbm, h_hbm,
             s_v, src_v, svals_v, idx_v, rows_v, hsum_v, sems, sems_w,
             sems_src):
    cid = lax.axis_index("c")
    sid = lax.axis_index("s")
    wid = sid * 2 + cid
    iota = lax.iota(jnp.int32, GROUP)
    ioff = iota * DEG
    minf = jnp.full((GROUP,), -jnp.inf, jnp.float32)

    CH = GROUP * DEG

    def _chunk_copy(b, g):
        return pltpu.make_async_copy(ei_hbm.at[0, pl.ds(g * CH, CH)],
                                     src_v[b], sems_src[b])

    _chunk_copy(0, wid).start()
    pltpu.sync_copy(s_hbm.at[0], s_v)

    def select_and_fire(b, g):
        _chunk_copy(b, g).wait()
        gn = g + NWORKERS

        @pl.when(gn < NGROUPS)
        def _():
            _chunk_copy(1 - b, gn).start()


        zero_ai = jnp.zeros((GROUP,), jnp.int32)

        def fillmax(j, carry):
            m, ai = carry
            srcs = plsc.load_gather(src_v[b], [ioff + j])
            sv = plsc.load_gather(s_v, [srcs])
            svals_v[pl.ds(j * GROUP, GROUP)] = sv
            gt = sv > m
            return jnp.where(gt, sv, m), jnp.where(gt, j, ai)
        m, ai = lax.fori_loop(0, DEG, fillmax, (minf, zero_ai), unroll=8)

        for p in range(K):
            plsc.store_scatter(svals_v, [ai * GROUP + iota], minf)
            srcsel = plsc.load_gather(src_v[b], [ioff + ai])
            idx_v[b][pl.ds(p * GROUP, GROUP)] = srcsel
            if p == K - 1:
                break

            def scanmax(j, carry):
                m, ai = carry
                v = svals_v[pl.ds(j * GROUP, GROUP)]
                gt = v > m
                return jnp.where(gt, v, m), jnp.where(gt, j, ai)
            m, ai = lax.fori_loop(0, DEG, scanmax, (minf, zero_ai),
                                  unroll=8)

        pltpu.make_async_copy(f_hbm.at[idx_v[b]], rows_v[b], sems[b]).start()

    def drain(b, g):
        pltpu.make_async_copy(f_hbm.at[idx_v[b]], rows_v[b], sems[b]).wait()

        @pl.when(g >= 2 * NWORKERS)
        def _():
            pltpu.make_async_copy(
                hsum_v[b], h_hbm.at[pl.ds((g - 2 * NWORKERS) * GROUP, GROUP)],
                sems_w[b]).wait()

        @plsc.parallel_loop(0, GROUP)
        def _(l):
            for d in range(D // 16):
                acc = rows_v[b][l, pl.ds(d * 16, 16)]
                for p in range(1, K):
                    acc = acc + rows_v[b][p * GROUP + l, pl.ds(d * 16, 16)]
                hsum_v[b][l, pl.ds(d * 16, 16)] = acc

        pltpu.make_async_copy(hsum_v[b],
                              h_hbm.at[pl.ds(g * GROUP, GROUP)],
                              sems_w[b]).start()

    def step(k, c):
        for bb in range(2):
            i = k * 2 + bb
            g = wid + i * NWORKERS

            @pl.when(g < NGROUPS)
            def _():
                select_and_fire(bb, g)

            gp = wid + (i - 1) * NWORKERS

            @pl.when(jnp.logical_and(i >= 1, gp < NGROUPS))
            def _():
                drain(1 - bb, gp)
        return c

    lax.fori_loop(0, MAXI // 2, step, 0)
    g_last = wid + (MAXI - 1) * NWORKERS

    @pl.when(g_last < NGROUPS)
    def _():
        drain((MAXI - 1) % 2, g_last)

    for b in range(2):
        pltpu.make_async_copy(hsum_v[b], h_hbm.at[pl.ds(0, GROUP)],
                              sems_w[b]).wait()


def _sc_gather(features, s2d, edge_index):
    mesh = plsc.VectorSubcoreMesh(core_axis_name="c", subcore_axis_name="s")
    run = pl.kernel(
        _sc_body,
        out_type=jax.ShapeDtypeStruct((N, D), jnp.float32),
        mesh=mesh,
        scratch_types=[
            pltpu.VMEM((N,), jnp.float32),
            [pltpu.VMEM((GROUP * DEG,), jnp.int32)] * 2,
            pltpu.VMEM((DEG * GROUP,), jnp.float32),
            [pltpu.VMEM((K * GROUP,), jnp.int32)] * 2,
            [pltpu.VMEM((K * GROUP, D), jnp.float32)] * 2,
            [pltpu.VMEM((GROUP, D), jnp.float32)] * 2,
            [pltpu.SemaphoreType.DMA] * 2,
            [pltpu.SemaphoreType.DMA] * 2,
            [pltpu.SemaphoreType.DMA] * 2,
        ],
        compiler_params=pltpu.CompilerParams(needs_layout_passes=False),
    )
    return run(features, s2d, edge_index)


def _out_body(f_ref, h_ref, w_ref, b_ref, o_ref):
    x = (f_ref[...] + h_ref[...] * (1.0 / K)).astype(jnp.bfloat16)
    wb = w_ref[...].astype(jnp.bfloat16)
    o_ref[...] = lax.dot_general(x, wb, (((1,), (1,)), ((), ())),
                                 preferred_element_type=jnp.float32) + b_ref[...]


def _tc_out(features, h, W_self, b_row):
    return pl.pallas_call(
        _out_body,
        grid=(N // ROW_BLK,),
        in_specs=[
            pl.BlockSpec((ROW_BLK, D), lambda i: (i, 0)),
            pl.BlockSpec((ROW_BLK, D), lambda i: (i, 0)),
            pl.BlockSpec((D, D), lambda i: (0, 0)),
            pl.BlockSpec((1, D), lambda i: (0, 0)),
        ],
        out_specs=pl.BlockSpec((ROW_BLK, D), lambda i: (i, 0)),
        out_shape=jax.ShapeDtypeStruct((N, D), jnp.float32),
    )(features, h, W_self, b_row)


@jax.jit
def _run(features, edge_index, W_at, W_atat, W_self, b_self):
    s = _tc_score(features, W_at, W_atat)
    h = _sc_gather(features, s, edge_index)
    return _tc_out(features, h, W_self, b_self.reshape(1, D))


def kernel(features, edge_index, PERSON_NUM, JOINT_NUM,
           W_at, W_atat, W_self, b_self):
    return _run(features, edge_index.astype(jnp.int32),
                W_at, W_atat, W_self, b_self)
